# baseline (speedup 1.0000x reference)
import functools

import jax
import jax.numpy as jnp
from jax import lax
from jax.experimental import pallas as pl
from jax.experimental.pallas import tpu as pltpu
from jax.experimental.pallas import tpu_sc as plsc

F32 = jnp.float32
EPS = 1e-5

NC = 2
NS = 16
NW = NC * NS
K = 80


def _zero_acc(buf, acc, s, rows_per_tile, d):
    def body(r, _):
        for j in range(d // 16):
            buf[0, r, pl.ds(j * 16, 16)] = jnp.zeros((16,), F32)
        return 0
    lax.fori_loop(0, K, body, 0)
    for j in range(rows_per_tile // K):
        pltpu.sync_copy(buf.at[0], acc.at[pl.ds(s * rows_per_tile + j * K, K)])


def _sc_gather_body(n_pad, d, ew, atom_hbm, src_hbm, dst_hbm,
                    sumh_hbm, part_hbm, sidx, didx, buf_a, buf_b, acc, sem):
    c = lax.axis_index("c")
    s = lax.axis_index("s")
    wid = s * NC + c
    rows_per_tile = n_pad // NS

    _zero_acc(buf_a, acc, s, rows_per_tile, d)
    plsc.subcore_barrier()

    nc_chunks = ew // K
    blk_rows = sidx.shape[0]
    base_row = wid * nc_chunks

    a = (buf_a.at[0], buf_a.at[1])
    bbuf = (buf_b.at[0], buf_b.at[1])

    def fire(j, sl):
        pltpu.async_copy(atom_hbm.at[sidx.at[j]], a[sl], sem)
        pltpu.async_copy(atom_hbm.at[didx.at[j]], bbuf[sl], sem)

    def process(i, j, sl):
        pltpu.make_async_copy(sumh_hbm.at[pl.ds(0, K)], a[sl], sem).wait()
        pltpu.make_async_copy(sumh_hbm.at[pl.ds(0, K)], bbuf[sl], sem).wait()
        av, bv = a[sl], bbuf[sl]

        def addrow(r, _):
            for jj in range(d // 16):
                s_ = pl.ds(jj * 16, 16)
                bv[r, s_] = bv[r, s_] + av[r, s_]
            return 0
        lax.fori_loop(0, K, addrow, 0)

        off = (base_row + i) * K
        pltpu.sync_copy(bv, sumh_hbm.at[pl.ds(off, K)])
        pltpu.sync_copy(av, acc.at[didx.at[j]], add=True)

    for blk in range((nc_chunks + blk_rows - 1) // blk_rows):
        nrows = min(blk_rows, nc_chunks - blk * blk_rows)
        cbase = blk * blk_rows
        pltpu.sync_copy(src_hbm.at[wid, pl.ds(cbase, blk_rows)], sidx)
        pltpu.sync_copy(dst_hbm.at[wid, pl.ds(cbase, blk_rows)], didx)
        fire(0, 0)

        def pair(p, _):
            j1 = 2 * p + 1
            fire(j1, 1)
            process(cbase + j1 - 1, j1 - 1, 0)

            @pl.when(j1 + 1 < nrows)
            def _():
                fire(j1 + 1, 0)
            process(cbase + j1, j1, 1)
            return 0

        lax.fori_loop(0, nrows // 2, pair, 0)
        if nrows % 2 == 1:
            process(cbase + nrows - 1, nrows - 1, 0)
    plsc.subcore_barrier()
    for j in range(rows_per_tile // K):
        r0 = s * rows_per_tile + j * K
        pltpu.sync_copy(acc.at[pl.ds(r0, K)], part_hbm.at[c, pl.ds(r0, K)])


def _sc_esum_body(n_pad, d, ew, e_hbm, dst_hbm, part_hbm,
                  didx, buf_a, acc, sem):
    c = lax.axis_index("c")
    s = lax.axis_index("s")
    wid = s * NC + c
    rows_per_tile = n_pad // NS

    _zero_acc(buf_a, acc, s, rows_per_tile, d)
    plsc.subcore_barrier()

    base = wid * ew
    nc_chunks = ew // K

    pltpu.sync_copy(dst_hbm.at[pl.ds(base, K)], didx.at[0])
    pltpu.async_copy(e_hbm.at[pl.ds(base, K)], buf_a.at[0], sem)

    def chunk(i, _):
        b = i % 2
        nb = 1 - b

        @pl.when(i + 1 < nc_chunks)
        def _prefetch():
            off_n = base + (i + 1) * K
            pltpu.sync_copy(dst_hbm.at[pl.ds(off_n, K)], didx.at[nb])
            pltpu.async_copy(e_hbm.at[pl.ds(off_n, K)], buf_a.at[nb], sem)

        off = base + i * K
        pltpu.make_async_copy(e_hbm.at[pl.ds(off, K)], buf_a.at[b], sem).wait()
        pltpu.sync_copy(buf_a.at[b], acc.at[didx.at[b]], add=True)
        return 0

    lax.fori_loop(0, nc_chunks, chunk, 0)
    plsc.subcore_barrier()
    for j in range(rows_per_tile // K):
        r0 = s * rows_per_tile + j * K
        pltpu.sync_copy(acc.at[pl.ds(r0, K)], part_hbm.at[c, pl.ds(r0, K)])


def _tc_edge1_body(sumh, bond, wt, wb, b1, y_out, st_out):
    y = jnp.dot(sumh[...], wt[...], preferred_element_type=F32)
    y = y + jnp.dot(bond[...], wb[...], preferred_element_type=F32)
    y = y + b1[...]
    y_out[...] = y.astype(jnp.bfloat16)
    s1 = jnp.sum(y, axis=0, keepdims=True)
    s2 = jnp.sum(y * y, axis=0, keepdims=True)
    st = jnp.concatenate([s1, s2], axis=0)

    @pl.when(pl.program_id(0) == 0)
    def _init():
        st_out[...] = jnp.zeros_like(st_out)

    st_out[...] += st


def _tc_edge1b_body(y_prev, sumh, bond, wt, wb, b1, y_out, st_out):
    del y_prev
    _tc_edge1_body(sumh, bond, wt, wb, b1, y_out, st_out)


def _tc_edge2_body(n_edges, y_in, st_a, st_b, g, bt, w2, b2, e_out):
    inv_e = 1.0 / n_edges
    st = st_a[...] + st_b[...]
    mu = st[0:1, :] * inv_e
    var = st[1:2, :] * inv_e - mu * mu
    a = g[...] * lax.rsqrt(var + EPS)
    cb = bt[...] - mu * a
    t = jnp.maximum(y_in[...].astype(F32) * a + cb, 0.0)
    e_out[...] = jnp.dot(t, w2[...], preferred_element_type=F32) + b2[...]


def _tc_node_body(n_nodes, ph_a, ph_b, pe, wt, wb, b1, g, bt, w2, b2, h_out):
    hn = ph_a[0] + ph_a[1] + ph_b[0] + ph_b[1]
    en = pe[0] + pe[1]
    y = jnp.dot(hn, wt[...], preferred_element_type=F32)
    y = y + jnp.dot(en, wb[...], preferred_element_type=F32)
    y = y + b1[...]
    mu = jnp.mean(y, axis=0, keepdims=True)
    var = jnp.mean(y * y, axis=0, keepdims=True) - mu * mu
    t = jnp.maximum((y - mu) * lax.rsqrt(var + EPS) * g[...] + bt[...], 0.0)
    h_out[...] = jnp.dot(t, w2[...], preferred_element_type=F32) + b2[...]


def kernel(atom, bond, edge_index, Wa1, ba1, ga, bta, Wa2, ba2,
           Wb1, bb1, gb, btb, Wb2, bb2):
    n, d = atom.shape
    e_cnt, _ = bond.shape
    h_dim = Wb1.shape[1]
    ew = e_cnt // NW
    src = edge_index[0]
    dst = edge_index[1]

    mesh = plsc.VectorSubcoreMesh(core_axis_name="c", subcore_axis_name="s",
                                  num_cores=NC, num_subcores=NS)
    n_pad = 10240

    blk_rows = 32
    blk = 2560
    nblk = e_cnt // blk
    nblk_a = nblk // 2
    e_a = nblk_a * blk
    e_b = e_cnt - e_a

    def _idx3(v, lo, hi):
        nch = (hi - lo) // (NW * K)
        rows_pad = ((nch + blk_rows - 1) // blk_rows) * blk_rows
        v = v[lo:hi].reshape(NW, nch, K)
        return jnp.pad(v, ((0, 0), (0, rows_pad - nch), (0, 0)))

    def _gather_half(lo, hi):
        ew_h = (hi - lo) // NW
        call = pl.kernel(
            functools.partial(_sc_gather_body, n_pad, d, ew_h),
            out_type=(jax.ShapeDtypeStruct((hi - lo, d), F32),
                      jax.ShapeDtypeStruct((NC, n_pad, d), F32)),
            mesh=mesh,
            scratch_types=[
                pltpu.VMEM((blk_rows, K), jnp.int32),
                pltpu.VMEM((blk_rows, K), jnp.int32),
                pltpu.VMEM((2, K, d), F32),
                pltpu.VMEM((2, K, d), F32),
                pltpu.VMEM_SHARED((n_pad, d), F32),
                pltpu.SemaphoreType.DMA,
            ],
        )
        return call(atom, _idx3(src, lo, hi), _idx3(dst, lo, hi))

    sumh_a, part_h_a = _gather_half(0, e_a)
    sumh_b, part_h_b = _gather_half(e_a, e_cnt)

    w_specs = [
        pl.BlockSpec((d, h_dim), lambda i: (0, 0)),
        pl.BlockSpec((d, h_dim), lambda i: (0, 0)),
        pl.BlockSpec((1, h_dim), lambda i: (0, 0)),
    ]
    w_args = (Wb1[:d], Wb1[d:], bb1.reshape(1, h_dim))

    y0, stats_a = pl.pallas_call(
        _tc_edge1_body,
        grid=(nblk_a,),
        in_specs=[
            pl.BlockSpec((blk, d), lambda i: (i, 0)),
            pl.BlockSpec((blk, d), lambda i: (i, 0)),
            *w_specs,
        ],
        out_specs=[
            pl.BlockSpec((blk, h_dim), lambda i: (i, 0)),
            pl.BlockSpec((2, h_dim), lambda i: (0, 0)),
        ],
        out_shape=[
            jax.ShapeDtypeStruct((e_cnt, h_dim), jnp.bfloat16),
            jax.ShapeDtypeStruct((2, h_dim), F32),
        ],
    )(sumh_a, bond, *w_args)

    na = nblk_a
    y, stats_b = pl.pallas_call(
        _tc_edge1b_body,
        grid=(nblk - nblk_a,),
        in_specs=[
            pl.BlockSpec((blk, h_dim), lambda i: (i + na, 0)),
            pl.BlockSpec((blk, d), lambda i: (i, 0)),
            pl.BlockSpec((blk, d), lambda i: (i + na, 0)),
            *w_specs,
        ],
        out_specs=[
            pl.BlockSpec((blk, h_dim), lambda i: (i + na, 0)),
            pl.BlockSpec((2, h_dim), lambda i: (0, 0)),
        ],
        out_shape=[
            jax.ShapeDtypeStruct((e_cnt, h_dim), jnp.bfloat16),
            jax.ShapeDtypeStruct((2, h_dim), F32),
        ],
        input_output_aliases={0: 0},
    )(y0, sumh_b, bond, *w_args)

    e_out = pl.pallas_call(
        functools.partial(_tc_edge2_body, float(e_cnt)),
        grid=(nblk,),
        in_specs=[
            pl.BlockSpec((blk, h_dim), lambda i: (i, 0)),
            pl.BlockSpec((2, h_dim), lambda i: (0, 0)),
            pl.BlockSpec((2, h_dim), lambda i: (0, 0)),
            pl.BlockSpec((1, h_dim), lambda i: (0, 0)),
            pl.BlockSpec((1, h_dim), lambda i: (0, 0)),
            pl.BlockSpec((h_dim, d), lambda i: (0, 0)),
            pl.BlockSpec((1, d), lambda i: (0, 0)),
        ],
        out_specs=pl.BlockSpec((blk, d), lambda i: (i, 0)),
        out_shape=jax.ShapeDtypeStruct((e_cnt, d), F32),
    )(y, stats_a, stats_b, gb.reshape(1, h_dim), btb.reshape(1, h_dim),
      Wb2, bb2.reshape(1, d))

    esum_call = pl.kernel(
        functools.partial(_sc_esum_body, n_pad, d, ew),
        out_type=jax.ShapeDtypeStruct((NC, n_pad, d), F32),
        mesh=mesh,
        scratch_types=[
            pltpu.VMEM((2, K), jnp.int32),
            pltpu.VMEM((2, K, d), F32),
            pltpu.VMEM_SHARED((n_pad, d), F32),
            pltpu.SemaphoreType.DMA,
        ],
    )
    part_e = esum_call(e_out, dst)

    h = pl.pallas_call(
        functools.partial(_tc_node_body, float(n)),
        grid=(1,),
        in_specs=[
            pl.BlockSpec((NC, n, d), lambda i: (0, 0, 0)),
            pl.BlockSpec((NC, n, d), lambda i: (0, 0, 0)),
            pl.BlockSpec((NC, n, d), lambda i: (0, 0, 0)),
            pl.BlockSpec((d, h_dim), lambda i: (0, 0)),
            pl.BlockSpec((d, h_dim), lambda i: (0, 0)),
            pl.BlockSpec((1, h_dim), lambda i: (0, 0)),
            pl.BlockSpec((1, h_dim), lambda i: (0, 0)),
            pl.BlockSpec((1, h_dim), lambda i: (0, 0)),
            pl.BlockSpec((h_dim, d), lambda i: (0, 0)),
            pl.BlockSpec((1, d), lambda i: (0, 0)),
        ],
        out_specs=pl.BlockSpec((n, d), lambda i: (0, 0)),
        out_shape=jax.ShapeDtypeStruct((n, d), F32),
    )(part_h_a, part_h_b, part_e, Wa1[:d], Wa1[d:], ba1.reshape(1, h_dim),
      ga.reshape(1, h_dim), bta.reshape(1, h_dim), Wa2, ba2.reshape(1, d))

    return h, e_out


# speedup vs baseline: 1.0391x; 1.0391x over previous
import functools

import jax
import jax.numpy as jnp
from jax import lax
from jax.experimental import pallas as pl
from jax.experimental.pallas import tpu as pltpu
from jax.experimental.pallas import tpu_sc as plsc

F32 = jnp.float32
EPS = 1e-5

NC = 2
NS = 16
NW = NC * NS
K = 80


def _zero_acc(buf, acc, s, rows_per_tile, d):
    def body(r, _):
        for j in range(d // 16):
            buf[0, r, pl.ds(j * 16, 16)] = jnp.zeros((16,), F32)
        return 0
    lax.fori_loop(0, K, body, 0)
    for j in range(rows_per_tile // K):
        pltpu.sync_copy(buf.at[0], acc.at[pl.ds(s * rows_per_tile + j * K, K)])


def _sc_gather_body(n_pad, d, ew, atom_hbm, src_hbm, dst_hbm,
                    sumh_hbm, part_hbm, sidx, didx, buf_a, buf_b, acc, sem,
                    semw0, semw1):
    c = lax.axis_index("c")
    s = lax.axis_index("s")
    wid = s * NC + c
    rows_per_tile = n_pad // NS

    _zero_acc(buf_a, acc, s, rows_per_tile, d)
    plsc.subcore_barrier()

    nc_chunks = ew // K
    blk_rows = sidx.shape[0]
    base_row = wid * nc_chunks

    a = (buf_a.at[0], buf_a.at[1])
    bbuf = (buf_b.at[0], buf_b.at[1])
    semw = (semw0, semw1)

    def fire(g, j, sl):
        @pl.when(g >= 2)
        def _():
            pltpu.make_async_copy(sumh_hbm.at[pl.ds(0, K)], bbuf[sl],
                                  semw[sl]).wait()
        pltpu.async_copy(atom_hbm.at[sidx.at[j]], a[sl], sem)
        pltpu.async_copy(atom_hbm.at[didx.at[j]], bbuf[sl], sem)

    def process(i, j, sl):
        pltpu.make_async_copy(sumh_hbm.at[pl.ds(0, K)], a[sl], sem).wait()
        pltpu.make_async_copy(sumh_hbm.at[pl.ds(0, K)], bbuf[sl], sem).wait()
        av, bv = a[sl], bbuf[sl]

        def addrow(r, _):
            for jj in range(d // 16):
                s_ = pl.ds(jj * 16, 16)
                bv[r, s_] = bv[r, s_] + av[r, s_]
            return 0
        lax.fori_loop(0, K, addrow, 0)

        off = (base_row + i) * K
        pltpu.async_copy(bv, sumh_hbm.at[pl.ds(off, K)], semw[sl])
        pltpu.sync_copy(av, acc.at[didx.at[j]], add=True)

    for blk in range((nc_chunks + blk_rows - 1) // blk_rows):
        nrows = min(blk_rows, nc_chunks - blk * blk_rows)
        cbase = blk * blk_rows
        pltpu.sync_copy(src_hbm.at[wid, pl.ds(cbase, blk_rows)], sidx)
        pltpu.sync_copy(dst_hbm.at[wid, pl.ds(cbase, blk_rows)], didx)
        fire(cbase, 0, 0)

        def pair(p, _):
            j1 = 2 * p + 1
            fire(cbase + j1, j1, 1)
            process(cbase + j1 - 1, j1 - 1, 0)

            @pl.when(j1 + 1 < nrows)
            def _():
                fire(cbase + j1 + 1, j1 + 1, 0)
            process(cbase + j1, j1, 1)
            return 0

        lax.fori_loop(0, nrows // 2, pair, 0)
        if nrows % 2 == 1:
            process(cbase + nrows - 1, nrows - 1, 0)
    for sl in range(2):
        pltpu.make_async_copy(sumh_hbm.at[pl.ds(0, K)], bbuf[sl],
                              semw[sl]).wait()
    plsc.subcore_barrier()
    for j in range(rows_per_tile // K):
        r0 = s * rows_per_tile + j * K
        pltpu.sync_copy(acc.at[pl.ds(r0, K)], part_hbm.at[c, pl.ds(r0, K)])


def _sc_esum_body(n_pad, d, ew, lo, e_hbm, dst_hbm, part_hbm,
                  didx, buf_a, acc, sem, semw0, semw1, semw2):
    c = lax.axis_index("c")
    s = lax.axis_index("s")
    wid = s * NC + c
    rows_per_tile = n_pad // NS

    _zero_acc(buf_a, acc, s, rows_per_tile, d)
    plsc.subcore_barrier()

    base = lo + wid * ew
    nch = ew // K
    bufs = (buf_a.at[0], buf_a.at[1], buf_a.at[2])
    idxs = (didx.at[0], didx.at[1], didx.at[2])
    semw = (semw0, semw1, semw2)

    def fire(g, sl):
        @pl.when(g >= 3)
        def _():
            pltpu.make_async_copy(e_hbm.at[pl.ds(0, K)], bufs[sl],
                                  semw[sl]).wait()
        pltpu.sync_copy(dst_hbm.at[pl.ds(base + g * K, K)], idxs[sl])
        pltpu.async_copy(e_hbm.at[pl.ds(base + g * K, K)], bufs[sl], sem)

    def process(g, sl):
        pltpu.make_async_copy(e_hbm.at[pl.ds(0, K)], bufs[sl], sem).wait()
        pltpu.async_copy(bufs[sl], acc.at[idxs[sl]], semw[sl], add=True)

    fire(0, 0)

    def triple(t, _):
        g0 = 3 * t
        fire(g0 + 1, 1)
        process(g0, 0)
        fire(g0 + 2, 2)
        process(g0 + 1, 1)

        @pl.when(g0 + 3 < nch)
        def _():
            fire(g0 + 3, 0)
        process(g0 + 2, 2)
        return 0

    lax.fori_loop(0, nch // 3, triple, 0)
    rem = nch % 3
    rbase = nch - rem
    if rem >= 1:
        if rem == 2:
            fire(rbase + 1, (rbase + 1) % 3)
        process(rbase, rbase % 3)
        if rem == 2:
            process(rbase + 1, (rbase + 1) % 3)
    for sl in range(3):
        pltpu.make_async_copy(e_hbm.at[pl.ds(0, K)], bufs[sl], semw[sl]).wait()
    plsc.subcore_barrier()
    for j in range(rows_per_tile // K):
        r0 = s * rows_per_tile + j * K
        pltpu.sync_copy(acc.at[pl.ds(r0, K)], part_hbm.at[c, pl.ds(r0, K)])


def _tc_edge1_body(sumh, bond, wt, wb, b1, y_out, st_out):
    y = jnp.dot(sumh[...], wt[...], preferred_element_type=F32)
    y = y + jnp.dot(bond[...], wb[...], preferred_element_type=F32)
    y = y + b1[...]
    y_out[...] = y.astype(jnp.bfloat16)
    s1 = jnp.sum(y, axis=0, keepdims=True)
    s2 = jnp.sum(y * y, axis=0, keepdims=True)
    st = jnp.concatenate([s1, s2], axis=0)

    @pl.when(pl.program_id(0) == 0)
    def _init():
        st_out[...] = jnp.zeros_like(st_out)

    st_out[...] += st


def _tc_edge1b_body(y_prev, sumh, bond, wt, wb, b1, y_out, st_out):
    del y_prev
    _tc_edge1_body(sumh, bond, wt, wb, b1, y_out, st_out)


def _tc_edge2_body(n_edges, y_in, st_a, st_b, g, bt, w2, b2, e_out):
    inv_e = 1.0 / n_edges
    st = st_a[...] + st_b[...]
    mu = st[0:1, :] * inv_e
    var = st[1:2, :] * inv_e - mu * mu
    a = g[...] * lax.rsqrt(var + EPS)
    cb = bt[...] - mu * a
    t = jnp.maximum(y_in[...].astype(F32) * a + cb, 0.0)
    e_out[...] = jnp.dot(t, w2[...], preferred_element_type=F32) + b2[...]


def _tc_node_body(n_nodes, ph_a, ph_b, pe, wt, wb, b1, g, bt, w2, b2, h_out):
    hn = ph_a[0] + ph_a[1] + ph_b[0] + ph_b[1]
    en = pe[0] + pe[1]
    y = jnp.dot(hn, wt[...], preferred_element_type=F32)
    y = y + jnp.dot(en, wb[...], preferred_element_type=F32)
    y = y + b1[...]
    mu = jnp.mean(y, axis=0, keepdims=True)
    var = jnp.mean(y * y, axis=0, keepdims=True) - mu * mu
    t = jnp.maximum((y - mu) * lax.rsqrt(var + EPS) * g[...] + bt[...], 0.0)
    h_out[...] = jnp.dot(t, w2[...], preferred_element_type=F32) + b2[...]


def kernel(atom, bond, edge_index, Wa1, ba1, ga, bta, Wa2, ba2,
           Wb1, bb1, gb, btb, Wb2, bb2):
    n, d = atom.shape
    e_cnt, _ = bond.shape
    h_dim = Wb1.shape[1]
    ew = e_cnt // NW
    src = edge_index[0]
    dst = edge_index[1]

    mesh = plsc.VectorSubcoreMesh(core_axis_name="c", subcore_axis_name="s",
                                  num_cores=NC, num_subcores=NS)
    n_pad = 10240

    blk_rows = 32
    blk = 2560
    nblk = e_cnt // blk
    nblk_a = nblk // 2
    e_a = nblk_a * blk
    e_b = e_cnt - e_a

    def _idx3(v, lo, hi):
        nch = (hi - lo) // (NW * K)
        rows_pad = ((nch + blk_rows - 1) // blk_rows) * blk_rows
        v = v[lo:hi].reshape(NW, nch, K)
        return jnp.pad(v, ((0, 0), (0, rows_pad - nch), (0, 0)))

    def _gather_half(lo, hi):
        ew_h = (hi - lo) // NW
        call = pl.kernel(
            functools.partial(_sc_gather_body, n_pad, d, ew_h),
            out_type=(jax.ShapeDtypeStruct((hi - lo, d), F32),
                      jax.ShapeDtypeStruct((NC, n_pad, d), F32)),
            mesh=mesh,
            scratch_types=[
                pltpu.VMEM((blk_rows, K), jnp.int32),
                pltpu.VMEM((blk_rows, K), jnp.int32),
                pltpu.VMEM((2, K, d), F32),
                pltpu.VMEM((2, K, d), F32),
                pltpu.VMEM_SHARED((n_pad, d), F32),
                pltpu.SemaphoreType.DMA,
                pltpu.SemaphoreType.DMA,
                pltpu.SemaphoreType.DMA,
            ],
        )
        return call(atom, _idx3(src, lo, hi), _idx3(dst, lo, hi))

    sumh_a, part_h_a = _gather_half(0, e_a)
    sumh_b, part_h_b = _gather_half(e_a, e_cnt)

    w_specs = [
        pl.BlockSpec((d, h_dim), lambda i: (0, 0)),
        pl.BlockSpec((d, h_dim), lambda i: (0, 0)),
        pl.BlockSpec((1, h_dim), lambda i: (0, 0)),
    ]
    w_args = (Wb1[:d], Wb1[d:], bb1.reshape(1, h_dim))

    y0, stats_a = pl.pallas_call(
        _tc_edge1_body,
        grid=(nblk_a,),
        in_specs=[
            pl.BlockSpec((blk, d), lambda i: (i, 0)),
            pl.BlockSpec((blk, d), lambda i: (i, 0)),
            *w_specs,
        ],
        out_specs=[
            pl.BlockSpec((blk, h_dim), lambda i: (i, 0)),
            pl.BlockSpec((2, h_dim), lambda i: (0, 0)),
        ],
        out_shape=[
            jax.ShapeDtypeStruct((e_cnt, h_dim), jnp.bfloat16),
            jax.ShapeDtypeStruct((2, h_dim), F32),
        ],
    )(sumh_a, bond, *w_args)

    na = nblk_a
    y, stats_b = pl.pallas_call(
        _tc_edge1b_body,
        grid=(nblk - nblk_a,),
        in_specs=[
            pl.BlockSpec((blk, h_dim), lambda i: (i + na, 0)),
            pl.BlockSpec((blk, d), lambda i: (i, 0)),
            pl.BlockSpec((blk, d), lambda i: (i + na, 0)),
            *w_specs,
        ],
        out_specs=[
            pl.BlockSpec((blk, h_dim), lambda i: (i + na, 0)),
            pl.BlockSpec((2, h_dim), lambda i: (0, 0)),
        ],
        out_shape=[
            jax.ShapeDtypeStruct((e_cnt, h_dim), jnp.bfloat16),
            jax.ShapeDtypeStruct((2, h_dim), F32),
        ],
        input_output_aliases={0: 0},
    )(y0, sumh_b, bond, *w_args)

    e_out = pl.pallas_call(
        functools.partial(_tc_edge2_body, float(e_cnt)),
        grid=(nblk,),
        in_specs=[
            pl.BlockSpec((blk, h_dim), lambda i: (i, 0)),
            pl.BlockSpec((2, h_dim), lambda i: (0, 0)),
            pl.BlockSpec((2, h_dim), lambda i: (0, 0)),
            pl.BlockSpec((1, h_dim), lambda i: (0, 0)),
            pl.BlockSpec((1, h_dim), lambda i: (0, 0)),
            pl.BlockSpec((h_dim, d), lambda i: (0, 0)),
            pl.BlockSpec((1, d), lambda i: (0, 0)),
        ],
        out_specs=pl.BlockSpec((blk, d), lambda i: (i, 0)),
        out_shape=jax.ShapeDtypeStruct((e_cnt, d), F32),
    )(y, stats_a, stats_b, gb.reshape(1, h_dim), btb.reshape(1, h_dim),
      Wb2, bb2.reshape(1, d))

    esum_call = pl.kernel(
        functools.partial(_sc_esum_body, n_pad, d, ew, 0),
        out_type=jax.ShapeDtypeStruct((NC, n_pad, d), F32),
        mesh=mesh,
        scratch_types=[
            pltpu.VMEM((3, K), jnp.int32),
            pltpu.VMEM((3, K, d), F32),
            pltpu.VMEM_SHARED((n_pad, d), F32),
            pltpu.SemaphoreType.DMA,
            pltpu.SemaphoreType.DMA,
            pltpu.SemaphoreType.DMA,
            pltpu.SemaphoreType.DMA,
        ],
    )
    part_e = esum_call(e_out, dst)

    h = pl.pallas_call(
        functools.partial(_tc_node_body, float(n)),
        grid=(1,),
        in_specs=[
            pl.BlockSpec((NC, n, d), lambda i: (0, 0, 0)),
            pl.BlockSpec((NC, n, d), lambda i: (0, 0, 0)),
            pl.BlockSpec((NC, n, d), lambda i: (0, 0, 0)),
            pl.BlockSpec((d, h_dim), lambda i: (0, 0)),
            pl.BlockSpec((d, h_dim), lambda i: (0, 0)),
            pl.BlockSpec((1, h_dim), lambda i: (0, 0)),
            pl.BlockSpec((1, h_dim), lambda i: (0, 0)),
            pl.BlockSpec((1, h_dim), lambda i: (0, 0)),
            pl.BlockSpec((h_dim, d), lambda i: (0, 0)),
            pl.BlockSpec((1, d), lambda i: (0, 0)),
        ],
        out_specs=pl.BlockSpec((n, d), lambda i: (0, 0)),
        out_shape=jax.ShapeDtypeStruct((n, d), F32),
    )(part_h_a, part_h_b, part_e, Wa1[:d], Wa1[d:], ba1.reshape(1, h_dim),
      ga.reshape(1, h_dim), bta.reshape(1, h_dim), Wa2, ba2.reshape(1, d))

    return h, e_out
